# baseline (speedup 1.0000x reference)
import functools

import jax
import jax.numpy as jnp
from jax import lax
from jax.experimental import pallas as pl
from jax.experimental.pallas import tpu as pltpu
from jax.experimental.pallas import tpu_sc as plsc

N_TOK = 4096
D_IN = 768
D_HID = 512
D_OUT = 768
N_EXP = 8
BLK = 512
N_BLK = N_TOK // BLK
LANES = 128
RROWS = BLK // LANES


def _shift_lanes_right(a, k):
    return jnp.concatenate([jnp.zeros((1, k), a.dtype), a[:, :-k]], axis=1)


def _router_rank_kernel(x_ref, wg_ref, bg_ref, rank_ref, hist_ref,
                        carry_scr, eid_scr, rank_scr):
    i = pl.program_id(0)

    @pl.when(i == 0)
    def _init():
        carry_scr[...] = jnp.zeros((1, LANES), jnp.float32)

    def _phase0():
        b = i
        logits8 = jnp.dot(x_ref[...], wg_ref[...],
                          preferred_element_type=jnp.float32) + bg_ref[...]
        logits = jnp.concatenate(
            [logits8, jnp.full((BLK, LANES - N_EXP), -1e30, jnp.float32)],
            axis=1)
        m = jnp.max(logits, axis=1, keepdims=True)
        lane = lax.broadcasted_iota(jnp.int32, (BLK, LANES), 1)
        eid = jnp.min(jnp.where(logits == m, lane, LANES), axis=1,
                      keepdims=True)
        onehot = (lane == eid).astype(jnp.float32)
        row = lax.broadcasted_iota(jnp.int32, (BLK, BLK), 0)
        col = lax.broadcasted_iota(jnp.int32, (BLK, BLK), 1)
        tril = (row >= col).astype(jnp.float32)
        within_incl = jnp.dot(tril, onehot,
                              preferred_element_type=jnp.float32)
        prefix = jnp.sum((within_incl + carry_scr[...]) * onehot,
                         axis=1, keepdims=True) - 1.0
        rank_scr[pl.ds(b * RROWS, RROWS), :] = (
            prefix.astype(jnp.int32).reshape(RROWS, LANES))
        eid_scr[pl.ds(b * RROWS, RROWS), :] = eid.reshape(RROWS, LANES)
        carry_scr[...] = carry_scr[...] + jnp.sum(onehot, axis=0,
                                                  keepdims=True)

    _phase0()

    @pl.when(i == N_BLK - 1)
    def _fixup():
        hist = carry_scr[...]
        incl = hist
        for k in (1, 2, 4):
            incl = incl + _shift_lanes_right(incl, k)
        start = incl - hist
        lane = lax.broadcasted_iota(jnp.int32, (1, LANES), 1)
        eid = eid_scr[...]
        corr = jnp.zeros((N_BLK * RROWS, LANES), jnp.int32)
        for e in range(N_EXP):
            s_e = jnp.sum(start * (lane == e).astype(jnp.float32),
                          axis=1, keepdims=True).astype(jnp.int32)
            corr = corr + jnp.where(eid == e, s_e, 0)
        rank_ref[...] = rank_scr[...] + corr

    hist_ref[...] = carry_scr[...].astype(jnp.int32)


def _router_rank(x, Wg, bg2d):
    return pl.pallas_call(
        _router_rank_kernel,
        grid=(N_BLK,),
        in_specs=[
            pl.BlockSpec((BLK, D_IN), lambda i: (i, 0)),
            pl.BlockSpec((D_IN, N_EXP), lambda i: (0, 0)),
            pl.BlockSpec((1, N_EXP), lambda i: (0, 0)),
        ],
        out_specs=[
            pl.BlockSpec((N_BLK * RROWS, LANES), lambda i: (0, 0)),
            pl.BlockSpec((1, LANES), lambda i: (0, 0)),
        ],
        out_shape=[
            jax.ShapeDtypeStruct((N_BLK * RROWS, LANES), jnp.int32),
            jax.ShapeDtypeStruct((1, LANES), jnp.int32),
        ],
        scratch_shapes=[
            pltpu.VMEM((1, LANES), jnp.float32),
            pltpu.VMEM((N_BLK * RROWS, LANES), jnp.int32),
            pltpu.VMEM((N_BLK * RROWS, LANES), jnp.int32),
        ],
    )(x, Wg, bg2d)


def _group_gemm_kernel(hist_ref, xs_ref, w1_ref, b1_ref, wout_ref,
                       bout_ref, out_ref, acc_scr, starts_scr):
    tile_base = pl.program_id(0) * BLK
    s = jnp.int32(0)
    e_lo = jnp.int32(0)
    e_hi = jnp.int32(0)
    starts_scr[0] = 0
    for e in range(N_EXP):
        e_lo = e_lo + jnp.where(s + hist_ref[e] <= tile_base, 1, 0)
        e_hi = e_hi + jnp.where(s < tile_base + BLK, 1, 0)
        s = s + hist_ref[e]
        starts_scr[e + 1] = s
    r = lax.broadcasted_iota(jnp.int32, (BLK, 1), 0) + tile_base

    def body(e, _):
        h = jnp.maximum(
            jnp.dot(xs_ref[...], w1_ref[e],
                    preferred_element_type=jnp.float32)
            + b1_ref[e][None, :], 0.0)
        mask = jnp.logical_and(r >= starts_scr[e], r < starts_scr[e + 1])
        acc_scr[...] = jnp.where(mask, h, acc_scr[...])
        return 0

    lax.fori_loop(e_lo, e_hi, body, 0)
    out_ref[...] = jnp.dot(acc_scr[...], wout_ref[...],
                           preferred_element_type=jnp.float32) + bout_ref[...]


def _group_gemm(hist128, xs, W1, b1, Wout, bout2d):
    return pl.pallas_call(
        _group_gemm_kernel,
        grid=(N_BLK,),
        in_specs=[
            pl.BlockSpec(memory_space=pltpu.SMEM),
            pl.BlockSpec((BLK, D_IN), lambda i: (i, 0)),
            pl.BlockSpec((N_EXP, D_IN, D_HID), lambda i: (0, 0, 0)),
            pl.BlockSpec((N_EXP, D_HID), lambda i: (0, 0)),
            pl.BlockSpec((D_HID, D_OUT), lambda i: (0, 0)),
            pl.BlockSpec((1, D_OUT), lambda i: (0, 0)),
        ],
        out_specs=pl.BlockSpec((BLK, D_OUT), lambda i: (i, 0)),
        out_shape=jax.ShapeDtypeStruct((N_TOK, D_OUT), jnp.float32),
        scratch_shapes=[pltpu.VMEM((BLK, D_HID), jnp.float32),
                        pltpu.SMEM((N_EXP + 1,), jnp.int32)],
    )(hist128, xs, W1, b1, Wout, bout2d)


_NC = 2
_NS = 16
_NW = _NC * _NS
_RPW = N_TOK // _NW


_NCH = 4
_CH = _RPW // _NCH


@functools.lru_cache(maxsize=None)
def _make_sc_permute(d_model, invert):
    mesh = plsc.VectorSubcoreMesh(core_axis_name="c", subcore_axis_name="s")

    @functools.partial(
        pl.kernel, mesh=mesh,
        out_type=jax.ShapeDtypeStruct((N_TOK, d_model), jnp.float32),
        scratch_types=(
            [pltpu.VMEM((_CH,), jnp.int32) for _ in range(_NCH)]
            + [pltpu.VMEM((_RPW, d_model), jnp.float32)]
            + [pltpu.SemaphoreType.DMA] * 4
        ),
    )
    def k(src_hbm, idx_hbm, out_hbm, i0, i1, i2, i3, rows_v,
          sem_i, sem_a, sem_b, sem_o):
        idx_refs = [i0, i1, i2, i3]
        sems = [sem_a, sem_b]
        wid = lax.axis_index("s") * _NC + lax.axis_index("c")
        base = wid * _RPW
        ics = [pltpu.async_copy(idx_hbm.at[pl.ds(base + c * _CH, _CH)],
                                idx_refs[c], sem_i) for c in range(_NCH)]

        def chunk(c):
            return rows_v.at[pl.ds(c * _CH, _CH)]

        rcs, wcs = [], []
        if invert:
            ics[0].wait()
            rcs.append(pltpu.async_copy(src_hbm.at[i0], chunk(0), sems[0]))
            for c in range(1, _NCH):
                ics[c].wait()
            for c in range(_NCH):
                rcs[c].wait()
                if c + 1 < _NCH:
                    rcs.append(pltpu.async_copy(
                        src_hbm.at[idx_refs[c + 1]], chunk(c + 1),
                        sems[(c + 1) % 2]))
                wcs.append(pltpu.async_copy(
                    chunk(c), out_hbm.at[pl.ds(base + c * _CH, _CH)], sem_o))
        else:
            rcs.append(pltpu.async_copy(
                src_hbm.at[pl.ds(base, _CH)], chunk(0), sems[0]))
            for c in range(_NCH):
                ics[c].wait()
            for c in range(_NCH):
                rcs[c].wait()
                if c + 1 < _NCH:
                    rcs.append(pltpu.async_copy(
                        src_hbm.at[pl.ds(base + (c + 1) * _CH, _CH)],
                        chunk(c + 1), sems[(c + 1) % 2]))
                wcs.append(pltpu.async_copy(
                    chunk(c), out_hbm.at[idx_refs[c]], sem_o))
        for w in wcs:
            w.wait()

    return k


def kernel(x, Wg, bg, W1, b1, Wout, bout):
    rank2d, hist2d = _router_rank(x, Wg, bg.reshape(1, N_EXP))
    rank = rank2d.reshape(N_TOK)
    hist128 = hist2d.reshape(LANES)
    xs = _make_sc_permute(D_IN, False)(x, rank)
    out_sorted = _group_gemm(hist128, xs, W1, b1, Wout,
                             bout.reshape(1, D_OUT))
    return _make_sc_permute(D_OUT, True)(out_sorted, rank)


# speedup vs baseline: 1.0664x; 1.0664x over previous
import functools

import jax
import jax.numpy as jnp
from jax import lax
from jax.experimental import pallas as pl
from jax.experimental.pallas import tpu as pltpu
from jax.experimental.pallas import tpu_sc as plsc

N_TOK = 4096
D_IN = 768
D_HID = 512
D_OUT = 768
N_EXP = 8
BLK = 512
N_BLK = N_TOK // BLK
LANES = 128
RROWS = BLK // LANES


def _shift_lanes_right(a, k):
    return jnp.concatenate([jnp.zeros((1, k), a.dtype), a[:, :-k]], axis=1)


def _router_rank_kernel(x_ref, wg_ref, bg_ref, rank_ref, hist_ref,
                        carry_scr, eid_scr, rank_scr):
    i = pl.program_id(0)

    @pl.when(i == 0)
    def _init():
        carry_scr[...] = jnp.zeros((1, LANES), jnp.float32)

    def _phase0():
        b = i
        logits8 = jnp.dot(x_ref[...], wg_ref[...],
                          preferred_element_type=jnp.float32) + bg_ref[...]
        logits = jnp.concatenate(
            [logits8, jnp.full((BLK, LANES - N_EXP), -1e30, jnp.float32)],
            axis=1)
        m = jnp.max(logits, axis=1, keepdims=True)
        lane = lax.broadcasted_iota(jnp.int32, (BLK, LANES), 1)
        eid = jnp.min(jnp.where(logits == m, lane, LANES), axis=1,
                      keepdims=True)
        onehot = (lane == eid).astype(jnp.float32)
        row = lax.broadcasted_iota(jnp.int32, (BLK, BLK), 0)
        col = lax.broadcasted_iota(jnp.int32, (BLK, BLK), 1)
        tril = (row >= col).astype(jnp.float32)
        within_incl = jnp.dot(tril, onehot,
                              preferred_element_type=jnp.float32)
        prefix = jnp.sum((within_incl + carry_scr[...]) * onehot,
                         axis=1, keepdims=True) - 1.0
        rank_scr[pl.ds(b * RROWS, RROWS), :] = (
            prefix.astype(jnp.int32).reshape(RROWS, LANES))
        eid_scr[pl.ds(b * RROWS, RROWS), :] = eid.reshape(RROWS, LANES)
        carry_scr[...] = carry_scr[...] + jnp.sum(onehot, axis=0,
                                                  keepdims=True)

    _phase0()

    @pl.when(i == N_BLK - 1)
    def _fixup():
        hist = carry_scr[...]
        incl = hist
        for k in (1, 2, 4):
            incl = incl + _shift_lanes_right(incl, k)
        start = incl - hist
        lane = lax.broadcasted_iota(jnp.int32, (1, LANES), 1)
        eid = eid_scr[...]
        corr = jnp.zeros((N_BLK * RROWS, LANES), jnp.int32)
        for e in range(N_EXP):
            s_e = jnp.sum(start * (lane == e).astype(jnp.float32),
                          axis=1, keepdims=True).astype(jnp.int32)
            corr = corr + jnp.where(eid == e, s_e, 0)
        rank_ref[...] = rank_scr[...] + corr

    hist_ref[...] = carry_scr[...].astype(jnp.int32)


def _router_rank(x, Wg, bg2d):
    return pl.pallas_call(
        _router_rank_kernel,
        grid=(N_BLK,),
        in_specs=[
            pl.BlockSpec((BLK, D_IN), lambda i: (i, 0)),
            pl.BlockSpec((D_IN, N_EXP), lambda i: (0, 0)),
            pl.BlockSpec((1, N_EXP), lambda i: (0, 0)),
        ],
        out_specs=[
            pl.BlockSpec((N_BLK * RROWS, LANES), lambda i: (0, 0)),
            pl.BlockSpec((1, LANES), lambda i: (0, 0)),
        ],
        out_shape=[
            jax.ShapeDtypeStruct((N_BLK * RROWS, LANES), jnp.int32),
            jax.ShapeDtypeStruct((1, LANES), jnp.int32),
        ],
        scratch_shapes=[
            pltpu.VMEM((1, LANES), jnp.float32),
            pltpu.VMEM((N_BLK * RROWS, LANES), jnp.int32),
            pltpu.VMEM((N_BLK * RROWS, LANES), jnp.int32),
        ],
    )(x, Wg, bg2d)


def _group_gemm_kernel(hist_ref, xs_ref, w1_ref, b1_ref, wout_ref,
                       bout_ref, out_ref, acc_scr, starts_scr):
    tile_base = pl.program_id(0) * BLK
    s = jnp.int32(0)
    e_lo = jnp.int32(0)
    e_hi = jnp.int32(0)
    starts_scr[0] = 0
    for e in range(N_EXP):
        e_lo = e_lo + jnp.where(s + hist_ref[e] <= tile_base, 1, 0)
        e_hi = e_hi + jnp.where(s < tile_base + BLK, 1, 0)
        s = s + hist_ref[e]
        starts_scr[e + 1] = s
    r = lax.broadcasted_iota(jnp.int32, (BLK, 1), 0) + tile_base

    def body(e, _):
        h = jnp.maximum(
            jnp.dot(xs_ref[...], w1_ref[e],
                    preferred_element_type=jnp.float32)
            + b1_ref[e][None, :], 0.0)
        mask = jnp.logical_and(r >= starts_scr[e], r < starts_scr[e + 1])
        acc_scr[...] = jnp.where(mask, h, acc_scr[...])
        return 0

    lax.fori_loop(e_lo, e_hi, body, 0)
    out_ref[...] = jnp.dot(acc_scr[...], wout_ref[...],
                           preferred_element_type=jnp.float32) + bout_ref[...]


def _group_gemm(hist128, xs, W1, b1, Wout, bout2d):
    return pl.pallas_call(
        _group_gemm_kernel,
        grid=(N_BLK,),
        in_specs=[
            pl.BlockSpec(memory_space=pltpu.SMEM),
            pl.BlockSpec((BLK, D_IN), lambda i: (i, 0)),
            pl.BlockSpec((N_EXP, D_IN, D_HID), lambda i: (0, 0, 0)),
            pl.BlockSpec((N_EXP, D_HID), lambda i: (0, 0)),
            pl.BlockSpec((D_HID, D_OUT), lambda i: (0, 0)),
            pl.BlockSpec((1, D_OUT), lambda i: (0, 0)),
        ],
        out_specs=pl.BlockSpec((BLK, D_OUT), lambda i: (i, 0)),
        out_shape=jax.ShapeDtypeStruct((N_TOK, D_OUT), jnp.float32),
        scratch_shapes=[pltpu.VMEM((BLK, D_HID), jnp.float32),
                        pltpu.SMEM((N_EXP + 1,), jnp.int32)],
    )(hist128, xs, W1, b1, Wout, bout2d)


_NC = 2
_NS = 16
_NW = _NC * _NS
_RPW = N_TOK // _NW


@functools.lru_cache(maxsize=None)
def _make_sc_permute(d_model, invert):
    mesh = plsc.VectorSubcoreMesh(core_axis_name="c", subcore_axis_name="s")

    @functools.partial(
        pl.kernel, mesh=mesh,
        out_type=jax.ShapeDtypeStruct((N_TOK, d_model), jnp.float32),
        scratch_types=[
            pltpu.VMEM((_RPW,), jnp.int32),
            pltpu.VMEM((_RPW, d_model), jnp.float32),
            pltpu.SemaphoreType.DMA,
        ],
    )
    def k(src_hbm, idx_hbm, out_hbm, idx_v, rows_v, sem):
        wid = lax.axis_index("s") * _NC + lax.axis_index("c")
        base = wid * _RPW
        pltpu.sync_copy(idx_hbm.at[pl.ds(base, _RPW)], idx_v)
        if invert:
            pltpu.async_copy(src_hbm.at[idx_v], rows_v, sem).wait()
            pltpu.sync_copy(rows_v, out_hbm.at[pl.ds(base, _RPW)])
        else:
            pltpu.sync_copy(src_hbm.at[pl.ds(base, _RPW)], rows_v)
            pltpu.async_copy(rows_v, out_hbm.at[idx_v], sem).wait()

    return k


def kernel(x, Wg, bg, W1, b1, Wout, bout):
    rank2d, hist2d = _router_rank(x, Wg, bg.reshape(1, N_EXP))
    rank = rank2d.reshape(N_TOK)
    hist128 = hist2d.reshape(LANES)
    xs = _make_sc_permute(D_IN, False)(x, rank)
    out_sorted = _group_gemm(hist128, xs, W1, b1, Wout,
                             bout.reshape(1, D_OUT))
    return _make_sc_permute(D_OUT, True)(out_sorted, rank)


# speedup vs baseline: 1.0732x; 1.0064x over previous
import functools

import jax
import jax.numpy as jnp
from jax import lax
from jax.experimental import pallas as pl
from jax.experimental.pallas import tpu as pltpu
from jax.experimental.pallas import tpu_sc as plsc

N_TOK = 4096
D_IN = 768
D_HID = 512
D_OUT = 768
N_EXP = 8
BLK = 512
N_BLK = N_TOK // BLK
LANES = 128
RROWS = BLK // LANES


def _shift_lanes_right(a, k):
    return jnp.concatenate([jnp.zeros((1, k), a.dtype), a[:, :-k]], axis=1)


def _router_rank_kernel(x_ref, wg_ref, bg_ref, rank_ref, hist_ref,
                        carry_scr, eid_scr, rank_scr):
    i = pl.program_id(0)

    @pl.when(i == 0)
    def _init():
        carry_scr[...] = jnp.zeros((1, LANES), jnp.float32)

    def _phase0():
        b = i
        logits8 = jnp.dot(x_ref[...], wg_ref[...],
                          preferred_element_type=jnp.float32) + bg_ref[...]
        logits = jnp.concatenate(
            [logits8, jnp.full((BLK, LANES - N_EXP), -1e30, jnp.float32)],
            axis=1)
        m = jnp.max(logits, axis=1, keepdims=True)
        lane = lax.broadcasted_iota(jnp.int32, (BLK, LANES), 1)
        eid = jnp.min(jnp.where(logits == m, lane, LANES), axis=1,
                      keepdims=True)
        onehot = (lane == eid).astype(jnp.float32)
        row = lax.broadcasted_iota(jnp.int32, (BLK, BLK), 0)
        col = lax.broadcasted_iota(jnp.int32, (BLK, BLK), 1)
        tril = (row >= col).astype(jnp.float32)
        within_incl = jnp.dot(tril, onehot,
                              preferred_element_type=jnp.float32)
        prefix = jnp.sum((within_incl + carry_scr[...]) * onehot,
                         axis=1, keepdims=True) - 1.0
        rank_scr[pl.ds(b * RROWS, RROWS), :] = (
            prefix.astype(jnp.int32).reshape(RROWS, LANES))
        eid_scr[pl.ds(b * RROWS, RROWS), :] = eid.reshape(RROWS, LANES)
        carry_scr[...] = carry_scr[...] + jnp.sum(onehot, axis=0,
                                                  keepdims=True)

    _phase0()

    @pl.when(i == N_BLK - 1)
    def _fixup():
        hist = carry_scr[...]
        incl = hist
        for k in (1, 2, 4):
            incl = incl + _shift_lanes_right(incl, k)
        start = incl - hist
        lane = lax.broadcasted_iota(jnp.int32, (1, LANES), 1)
        eid = eid_scr[...]
        corr = jnp.zeros((N_BLK * RROWS, LANES), jnp.int32)
        for e in range(N_EXP):
            s_e = jnp.sum(start * (lane == e).astype(jnp.float32),
                          axis=1, keepdims=True).astype(jnp.int32)
            corr = corr + jnp.where(eid == e, s_e, 0)
        rank_ref[...] = rank_scr[...] + corr

    hist_ref[...] = carry_scr[...].astype(jnp.int32)


def _router_rank(x, Wg, bg2d):
    return pl.pallas_call(
        _router_rank_kernel,
        grid=(N_BLK,),
        in_specs=[
            pl.BlockSpec((BLK, D_IN), lambda i: (i, 0)),
            pl.BlockSpec((D_IN, N_EXP), lambda i: (0, 0)),
            pl.BlockSpec((1, N_EXP), lambda i: (0, 0)),
        ],
        out_specs=[
            pl.BlockSpec((N_BLK * RROWS, LANES), lambda i: (0, 0)),
            pl.BlockSpec((1, LANES), lambda i: (0, 0)),
        ],
        out_shape=[
            jax.ShapeDtypeStruct((N_BLK * RROWS, LANES), jnp.int32),
            jax.ShapeDtypeStruct((1, LANES), jnp.int32),
        ],
        scratch_shapes=[
            pltpu.VMEM((1, LANES), jnp.float32),
            pltpu.VMEM((N_BLK * RROWS, LANES), jnp.int32),
            pltpu.VMEM((N_BLK * RROWS, LANES), jnp.int32),
        ],
    )(x, Wg, bg2d)


def _group_gemm_kernel(hist_ref, xs_ref, w1_ref, b1_ref, wout_ref,
                       bout_ref, out_ref, acc_scr, starts_scr):
    tile_base = pl.program_id(0) * BLK
    s = jnp.int32(0)
    e_lo = jnp.int32(0)
    e_hi = jnp.int32(0)
    starts_scr[0] = 0
    for e in range(N_EXP):
        e_lo = e_lo + jnp.where(s + hist_ref[e] <= tile_base, 1, 0)
        e_hi = e_hi + jnp.where(s < tile_base + BLK, 1, 0)
        s = s + hist_ref[e]
        starts_scr[e + 1] = s
    r = lax.broadcasted_iota(jnp.int32, (BLK, 1), 0) + tile_base

    def body(e, _):
        h = jnp.maximum(
            jnp.dot(xs_ref[...], w1_ref[e],
                    preferred_element_type=jnp.float32)
            + b1_ref[e][None, :], 0.0)
        mask = jnp.logical_and(r >= starts_scr[e], r < starts_scr[e + 1])
        acc_scr[...] = jnp.where(mask, h, acc_scr[...])
        return 0

    lax.fori_loop(e_lo, e_hi, body, 0)
    out_ref[...] = jnp.dot(acc_scr[...], wout_ref[...],
                           preferred_element_type=jnp.float32) + bout_ref[...]


def _group_gemm(hist128, xs, W1, b1, Wout, bout2d):
    return pl.pallas_call(
        _group_gemm_kernel,
        grid=(N_BLK,),
        in_specs=[
            pl.BlockSpec(memory_space=pltpu.SMEM),
            pl.BlockSpec((BLK, D_IN), lambda i: (i, 0)),
            pl.BlockSpec((N_EXP, D_IN, D_HID), lambda i: (0, 0, 0)),
            pl.BlockSpec((N_EXP, D_HID), lambda i: (0, 0)),
            pl.BlockSpec((D_HID, D_OUT), lambda i: (0, 0)),
            pl.BlockSpec((1, D_OUT), lambda i: (0, 0)),
        ],
        out_specs=pl.BlockSpec((BLK, D_OUT), lambda i: (i, 0)),
        out_shape=jax.ShapeDtypeStruct((N_TOK, D_OUT), jnp.float32),
        scratch_shapes=[pltpu.VMEM((BLK, D_HID), jnp.float32),
                        pltpu.SMEM((N_EXP + 1,), jnp.int32)],
    )(hist128, xs, W1, b1, Wout, bout2d)


_NC = 2
_NS = 16
_NW = _NC * _NS
_RPW = N_TOK // _NW


@functools.lru_cache(maxsize=None)
def _make_sc_permute(d_model, invert):
    mesh = plsc.VectorSubcoreMesh(core_axis_name="c", subcore_axis_name="s")

    half = _RPW // 2

    @functools.partial(
        pl.kernel, mesh=mesh,
        out_type=jax.ShapeDtypeStruct((N_TOK, d_model), jnp.float32),
        scratch_types=[
            pltpu.VMEM((half,), jnp.int32),
            pltpu.VMEM((half,), jnp.int32),
            pltpu.VMEM((_RPW, d_model), jnp.float32),
            pltpu.SemaphoreType.DMA,
            pltpu.SemaphoreType.DMA,
            pltpu.SemaphoreType.DMA,
        ],
    )
    def k(src_hbm, idx_hbm, out_hbm, i0, i1, rows_v, s0, s1, so):
        wid = lax.axis_index("s") * _NC + lax.axis_index("c")
        base = wid * _RPW
        idx = [i0, i1]
        sem = [s0, s1]
        ics = [pltpu.async_copy(idx_hbm.at[pl.ds(base + c * half, half)],
                                idx[c], so) for c in range(2)]

        def chunk(c):
            return rows_v.at[pl.ds(c * half, half)]

        wcs = []
        if invert:
            rcs = []
            for c in range(2):
                ics[c].wait()
                rcs.append(pltpu.async_copy(src_hbm.at[idx[c]], chunk(c),
                                            sem[c]))
            for c in range(2):
                rcs[c].wait()
                wcs.append(pltpu.async_copy(
                    chunk(c), out_hbm.at[pl.ds(base + c * half, half)], so))
        else:
            rcs = [pltpu.async_copy(src_hbm.at[pl.ds(base + c * half, half)],
                                    chunk(c), sem[c]) for c in range(2)]
            for c in range(2):
                ics[c].wait()
            for c in range(2):
                rcs[c].wait()
                wcs.append(pltpu.async_copy(chunk(c), out_hbm.at[idx[c]],
                                            so))
        for w in wcs:
            w.wait()

    return k


def kernel(x, Wg, bg, W1, b1, Wout, bout):
    rank2d, hist2d = _router_rank(x, Wg, bg.reshape(1, N_EXP))
    rank = rank2d.reshape(N_TOK)
    hist128 = hist2d.reshape(LANES)
    xs = _make_sc_permute(D_IN, False)(x, rank)
    out_sorted = _group_gemm(hist128, xs, W1, b1, Wout,
                             bout.reshape(1, D_OUT))
    return _make_sc_permute(D_OUT, True)(out_sorted, rank)
